# baseline (speedup 1.0000x reference)
import functools

import jax
import jax.numpy as jnp
from jax import lax
from jax.experimental import pallas as pl
from jax.experimental.pallas import tpu as pltpu

AUG_P = 0.5
DROP_P = 0.15
MIN_FEATURES = 1


def _mask_kernel(s_ref, m_ref, aug_ref, keep_ref, *, F, B):
    m = m_ref[...].T > 0
    bits = lax.bitcast_convert_type(s_ref[...].T, jnp.int32)
    bits = jnp.where(m, bits, jnp.int32(0x7FFFFFFF))

    n_avail = jnp.sum(m.astype(jnp.int32), axis=0, keepdims=True)
    k = (n_avail.astype(jnp.float32) * DROP_P).astype(jnp.int32)
    k = jnp.minimum(k, n_avail - MIN_FEATURES)
    aug = aug_ref[...].T < AUG_P
    k = jnp.where((n_avail > MIN_FEATURES) & aug & (k > 0), k, 0)

    ans = jnp.zeros((1, B), jnp.int32)
    for bit in range(30, -1, -1):
        test = ans + jnp.int32(1 << bit)
        cnt = jnp.sum((bits < test).astype(jnp.int32), axis=0, keepdims=True)
        ans = jnp.where(cnt < k, test, ans)

    c_lt = jnp.sum((bits < ans).astype(jnp.int32), axis=0, keepdims=True)
    eq = bits == ans
    fi = lax.broadcasted_iota(jnp.int32, (F, F), 0)
    fj = lax.broadcasted_iota(jnp.int32, (F, F), 1)
    tril = (fj < fi).astype(jnp.float32)
    eq_before = jax.lax.dot(
        tril, eq.astype(jnp.float32), precision=jax.lax.Precision.HIGHEST
    ).astype(jnp.int32)
    drop = m & ((bits < ans) | (eq & ((c_lt + eq_before) < k)))
    keep_ref[...] = (1.0 - drop.astype(jnp.float32)).T


def _copy_kernel(x_ref, k_ref, e_ref, o_ref):
    kw = jax.lax.dot(
        k_ref[...], e_ref[...], precision=jax.lax.Precision.HIGHEST
    )
    o_ref[...] = x_ref[...] * kw


def kernel(input_features, attention_mask):
    B, F, C = input_features.shape
    key = jax.random.key(42)
    k1, k2 = jax.random.split(key)
    aug_u = jax.random.uniform(k1, (B,)).reshape(B, 1)
    scores = jax.random.uniform(k2, (B, F))
    mask_i32 = attention_mask.astype(jnp.int32)

    keep = pl.pallas_call(
        functools.partial(_mask_kernel, F=F, B=B),
        out_shape=jax.ShapeDtypeStruct((B, F), jnp.float32),
    )(scores, mask_i32, aug_u)

    expand = jnp.kron(
        jnp.eye(F, dtype=jnp.float32), jnp.ones((1, C), jnp.float32)
    )
    x2 = input_features.reshape(B, F * C)

    BBLK = 64
    grid = (B // BBLK,)
    out = pl.pallas_call(
        _copy_kernel,
        grid=grid,
        compiler_params=pltpu.CompilerParams(
            dimension_semantics=("parallel",),
        ),
        in_specs=[
            pl.BlockSpec((BBLK, F * C), lambda i: (i, 0)),
            pl.BlockSpec((BBLK, F), lambda i: (i, 0)),
            pl.BlockSpec((F, F * C), lambda i: (0, 0)),
        ],
        out_specs=pl.BlockSpec((BBLK, F * C), lambda i: (i, 0)),
        out_shape=jax.ShapeDtypeStruct((B, F * C), input_features.dtype),
    )(x2, keep, expand)
    return out.reshape(B, F, C)


# speedup vs baseline: 1.0339x; 1.0339x over previous
import functools

import jax
import jax.numpy as jnp
from jax import lax
from jax.experimental import pallas as pl
from jax.experimental.pallas import tpu as pltpu
from jax.experimental.pallas import tpu_sc as plsc

AUG_P = 0.5
DROP_P = 0.15
MIN_FEATURES = 1
NLIST = 16


def _list_kernel(s_ref, m_ref, aug_ref, list_ref, *, F, B):
    m = m_ref[...].T > 0
    bits = lax.bitcast_convert_type(s_ref[...].T, jnp.int32)
    bits = jnp.where(m, bits, jnp.int32(0x7FFFFFFF))

    n_avail = jnp.sum(m.astype(jnp.int32), axis=0, keepdims=True)
    k = (n_avail.astype(jnp.float32) * DROP_P).astype(jnp.int32)
    k = jnp.minimum(k, n_avail - MIN_FEATURES)
    aug = aug_ref[...].T < AUG_P
    k = jnp.where((n_avail > MIN_FEATURES) & aug & (k > 0), k, 0)

    ans = jnp.zeros((1, B), jnp.int32)
    for bit in range(30, -1, -1):
        test = ans + jnp.int32(1 << bit)
        cnt = jnp.sum((bits < test).astype(jnp.int32), axis=0, keepdims=True)
        ans = jnp.where(cnt < k, test, ans)

    c_lt = jnp.sum((bits < ans).astype(jnp.int32), axis=0, keepdims=True)
    eq = bits == ans
    fi = lax.broadcasted_iota(jnp.int32, (F, F), 0)
    fj = lax.broadcasted_iota(jnp.int32, (F, F), 1)
    tril = (fj < fi).astype(jnp.float32)
    eq_before = jax.lax.dot(
        tril, eq.astype(jnp.float32), precision=jax.lax.Precision.HIGHEST
    ).astype(jnp.int32)
    drop = m & ((bits < ans) | (eq & ((c_lt + eq_before) < k)))

    dropf = drop.astype(jnp.float32)
    pos = jax.lax.dot(
        tril, dropf, precision=jax.lax.Precision.HIGHEST
    ).astype(jnp.int32)
    ff = lax.broadcasted_iota(jnp.int32, (F, B), 0)
    rows = []
    for kk in range(NLIST):
        sel = drop & (pos == kk)
        row = jnp.sum(jnp.where(sel, ff, 0), axis=0, keepdims=True)
        any_sel = jnp.sum(sel.astype(jnp.int32), axis=0, keepdims=True)
        rows.append(jnp.where(any_sel > 0, row, F))
    listT = jnp.concatenate(rows, axis=0)
    list_ref[...] = listT.T


def _sc_copy(x_hbm, dl_hbm, out_hbm, *scratch, B, F, C, NC, NW):
    RW = F * C
    ROWS_W = B // NW
    NBUF = 8
    NCHUNK = ROWS_W
    NROUND = NCHUNK // NBUF
    idx_v = scratch[0]
    bufs = scratch[1:1 + NBUF]
    sin = scratch[1 + NBUF:1 + 2 * NBUF]
    sout = scratch[1 + 2 * NBUF:1 + 3 * NBUF]

    wid = lax.axis_index("s") * NC + lax.axis_index("c")
    base = wid * ROWS_W
    pltpu.sync_copy(dl_hbm.at[pl.ds(base * NLIST, ROWS_W * NLIST)], idx_v)

    zeros16 = jnp.zeros((16,), jnp.float32)

    def in_desc(p, j):
        return pltpu.make_async_copy(
            x_hbm.at[pl.ds((base + p) * RW, RW)], bufs[j], sin[j]
        )

    def out_desc(p, j):
        return pltpu.make_async_copy(
            bufs[j], out_hbm.at[pl.ds((base + p) * RW, RW)], sout[j]
        )

    def zero_chunk(p, j):
        dl = idx_v[pl.ds(p * NLIST, NLIST)]
        for e in range(NLIST):
            off = dl[e]

            @pl.when(off < F)
            def _():
                o0 = off * C
                for q in range(C // 16):
                    bufs[j][pl.ds(o0 + q * 16, 16)] = zeros16

    for q in range(5):
        in_desc(q, q).start()

    def round_body(t, carry):
        for j in range(NBUF):
            p = t * NBUF + j

            @pl.when(p + 5 < NCHUNK)
            def _():
                @pl.when(p >= 3)
                def _():
                    out_desc(p - 3, (j + 5) % NBUF).wait()

                in_desc(p + 5, (j + 5) % NBUF).start()

            in_desc(p, j).wait()
            zero_chunk(p, j)
            out_desc(p, j).start()
        return carry

    lax.fori_loop(0, NROUND, round_body, 0)
    for q in range(NBUF):
        p = NCHUNK - NBUF + q
        out_desc(p, p % NBUF).wait()


def kernel(input_features, attention_mask):
    B, F, C = input_features.shape
    key = jax.random.key(42)
    k1, k2 = jax.random.split(key)
    aug_u = jax.random.uniform(k1, (B,)).reshape(B, 1)
    scores = jax.random.uniform(k2, (B, F))
    mask_i32 = attention_mask.astype(jnp.int32)

    dlist = pl.pallas_call(
        functools.partial(_list_kernel, F=F, B=B),
        out_shape=jax.ShapeDtypeStruct((B, NLIST), jnp.int32),
    )(scores, mask_i32, aug_u)

    info = plsc.get_sparse_core_info()
    NC, NS = info.num_cores, info.num_subcores
    NW = NC * NS
    RW = F * C
    mesh = plsc.VectorSubcoreMesh(core_axis_name="c", subcore_axis_name="s")
    body = functools.partial(_sc_copy, B=B, F=F, C=C, NC=NC, NW=NW)
    out = pl.kernel(
        body,
        mesh=mesh,
        out_type=jax.ShapeDtypeStruct((B * RW,), jnp.float32),
        scratch_types=(
            [pltpu.VMEM(((B // NW) * NLIST,), jnp.int32)]
            + [pltpu.VMEM((RW,), jnp.float32) for _ in range(8)]
            + [pltpu.SemaphoreType.DMA for _ in range(16)]
        ),
    )(input_features.reshape(B * RW), dlist.reshape(B * NLIST))
    return out.reshape(B, F, C)


# speedup vs baseline: 1.7883x; 1.7296x over previous
import functools

import jax
import jax.numpy as jnp
from jax import lax
from jax.experimental import pallas as pl
from jax.experimental.pallas import tpu as pltpu

AUG_P = 0.5
DROP_P = 0.15
MIN_FEATURES = 1


def _mask_kernel(s_ref, m_ref, aug_ref, keep_ref, *, F, B):
    m = m_ref[...].T > 0
    bits = lax.bitcast_convert_type(s_ref[...].T, jnp.int32)
    bits = jnp.where(m, bits, jnp.int32(0x7FFFFFFF))
    n_avail = jnp.sum(m.astype(jnp.int32), axis=0, keepdims=True)
    k = (n_avail.astype(jnp.float32) * DROP_P).astype(jnp.int32)
    k = jnp.minimum(k, n_avail - MIN_FEATURES)
    aug = aug_ref[...].T < AUG_P
    k = jnp.where((n_avail > MIN_FEATURES) & aug & (k > 0), k, 0)
    ans = jnp.zeros((1, B), jnp.int32)
    for bit in range(30, -1, -1):
        test = ans + jnp.int32(1 << bit)
        cnt = jnp.sum((bits < test).astype(jnp.int32), axis=0, keepdims=True)
        ans = jnp.where(cnt < k, test, ans)
    c_lt = jnp.sum((bits < ans).astype(jnp.int32), axis=0, keepdims=True)
    eq = bits == ans
    fi = lax.broadcasted_iota(jnp.int32, (F, F), 0)
    fj = lax.broadcasted_iota(jnp.int32, (F, F), 1)
    tril = (fj < fi).astype(jnp.float32)
    eq_before = jax.lax.dot(
        tril, eq.astype(jnp.float32), precision=jax.lax.Precision.HIGHEST
    ).astype(jnp.int32)
    drop = m & ((bits < ans) | (eq & ((c_lt + eq_before) < k)))
    keep_ref[...] = (1.0 - drop.astype(jnp.float32)).T


def _copy_big(x_hbm, keep_hbm, o_hbm, kv, b0, b1, ks, s0, s1, t0, t1,
              *, B, F, C, BBLK):
    NBLK = B // BBLK
    bufs = (b0, b1)
    sin = (s0, s1)
    sout = (t0, t1)

    pltpu.make_async_copy(keep_hbm, kv, ks).start()

    def in_desc(p, j):
        return pltpu.make_async_copy(
            x_hbm.at[pl.ds(p * BBLK, BBLK)], bufs[j], sin[j]
        )

    def out_desc(p, j):
        return pltpu.make_async_copy(
            bufs[j], o_hbm.at[pl.ds(p * BBLK, BBLK)], sout[j]
        )

    in_desc(0, 0).start()
    in_desc(1, 1).start()
    pltpu.make_async_copy(keep_hbm, kv, ks).wait()

    for p in range(NBLK):
        j = p % 2
        in_desc(p, j).wait()
        kb = kv[pl.ds(p * BBLK, BBLK), :]
        bufs[j][...] = bufs[j][...] * kb[:, :, None]
        out_desc(p, j).start()
        if p + 2 < NBLK:
            out_desc(p, j).wait()
            in_desc(p + 2, j).start()
    out_desc(NBLK - 2, (NBLK - 2) % 2).wait()
    out_desc(NBLK - 1, (NBLK - 1) % 2).wait()


def kernel(input_features, attention_mask):
    B, F, C = input_features.shape
    key = jax.random.key(42)
    k1, k2 = jax.random.split(key)
    aug_u = jax.random.uniform(k1, (B,)).reshape(B, 1)
    scores = jax.random.uniform(k2, (B, F))
    mask_i32 = attention_mask.astype(jnp.int32)

    keep = pl.pallas_call(
        functools.partial(_mask_kernel, F=F, B=B),
        out_shape=jax.ShapeDtypeStruct((B, F), jnp.float32),
    )(scores, mask_i32, aug_u)

    BBLK = 256
    out = pl.pallas_call(
        functools.partial(_copy_big, B=B, F=F, C=C, BBLK=BBLK),
        in_specs=[
            pl.BlockSpec(memory_space=pl.ANY),
            pl.BlockSpec(memory_space=pl.ANY),
        ],
        out_specs=pl.BlockSpec(memory_space=pl.ANY),
        out_shape=jax.ShapeDtypeStruct((B, F, C), input_features.dtype),
        scratch_shapes=(
            [pltpu.VMEM((B, F), jnp.float32)]
            + [pltpu.VMEM((BBLK, F, C), jnp.float32) for _ in range(2)]
            + [pltpu.SemaphoreType.DMA for _ in range(5)]
        ),
    )(input_features, keep)
    return out


# speedup vs baseline: 5.0108x; 2.8021x over previous
import functools

import jax
import jax.numpy as jnp
from jax import lax
from jax.experimental import pallas as pl
from jax.experimental.pallas import tpu as pltpu

AUG_P = 0.5
DROP_P = 0.15
MIN_FEATURES = 1
NRING = 5


def _fused(s_ref, m_ref, aug_ref, x_hbm, o_hbm, kv,
           i0, i1, i2, i3, i4, o0, o1, o2, o3, o4,
           si0, si1, si2, si3, si4, so0, so1, so2, so3, so4,
           *, B, F, C, FBLK):
    NBLK = F // FBLK
    NROUND = NBLK // NRING
    ibufs = (i0, i1, i2, i3, i4)
    obufs = (o0, o1, o2, o3, o4)
    sin = (si0, si1, si2, si3, si4)
    sout = (so0, so1, so2, so3, so4)

    def in_desc(p, j):
        return pltpu.make_async_copy(
            x_hbm.at[pl.ds(p * FBLK, FBLK)], ibufs[j], sin[j]
        )

    def out_desc(p, j):
        return pltpu.make_async_copy(
            obufs[j], o_hbm.at[pl.ds(p * FBLK, FBLK)], sout[j]
        )

    for q in range(NRING - 1):
        in_desc(q, q).start()

    m = m_ref[...].T > 0
    bits = lax.bitcast_convert_type(s_ref[...].T, jnp.int32)
    bits = jnp.where(m, bits, jnp.int32(0x7FFFFFFF))
    n_avail = jnp.sum(m.astype(jnp.int32), axis=0, keepdims=True)
    k = (n_avail.astype(jnp.float32) * DROP_P).astype(jnp.int32)
    k = jnp.minimum(k, n_avail - MIN_FEATURES)
    aug = aug_ref[...].T < AUG_P
    k = jnp.where((n_avail > MIN_FEATURES) & aug & (k > 0), k, 0)
    ans = jnp.zeros((1, B), jnp.int32)
    for bit in range(30, -1, -1):
        test = ans + jnp.int32(1 << bit)
        cnt = jnp.sum((bits < test).astype(jnp.int32), axis=0, keepdims=True)
        ans = jnp.where(cnt < k, test, ans)
    c_lt = jnp.sum((bits < ans).astype(jnp.int32), axis=0, keepdims=True)
    eq = bits == ans
    fi = lax.broadcasted_iota(jnp.int32, (F, F), 0)
    fj = lax.broadcasted_iota(jnp.int32, (F, F), 1)
    tril = (fj < fi).astype(jnp.float32)
    eq_before = jax.lax.dot(
        tril, eq.astype(jnp.float32), precision=jax.lax.Precision.HIGHEST
    ).astype(jnp.int32)
    drop = m & ((bits < ans) | (eq & ((c_lt + eq_before) < k)))
    kv[...] = 1.0 - drop.astype(jnp.float32)

    def round_body(t, carry):
        for j in range(NRING):
            p = t * NRING + j
            j2 = (j + NRING - 1) % NRING

            @pl.when(p + NRING - 1 < NBLK)
            def _():
                in_desc(p + NRING - 1, j2).start()

            in_desc(p, j).wait()

            @pl.when(t > 0)
            def _():
                out_desc(p - NRING, j).wait()

            sel = (
                lax.broadcasted_iota(jnp.int32, (FBLK, F), 1)
                == p * FBLK + lax.broadcasted_iota(jnp.int32, (FBLK, F), 0)
            ).astype(jnp.float32)
            kb = jax.lax.dot(
                sel, kv[...], precision=jax.lax.Precision.HIGHEST
            )
            obufs[j][...] = ibufs[j][...] * kb[:, :, None]
            out_desc(p, j).start()
        return carry

    lax.fori_loop(0, NROUND, round_body, 0)
    for j in range(NRING):
        out_desc(NBLK - NRING + j, j).wait()


def kernel(input_features, attention_mask):
    B, F, C = input_features.shape
    key = jax.random.key(42)
    k1, k2 = jax.random.split(key)
    aug_u = jax.random.uniform(k1, (B,)).reshape(B, 1)
    scores = jax.random.uniform(k2, (B, F))
    mask_i32 = attention_mask.astype(jnp.int32)

    xt = jnp.transpose(input_features, (1, 0, 2))

    FBLK = 5
    out_t = pl.pallas_call(
        functools.partial(_fused, B=B, F=F, C=C, FBLK=FBLK),
        in_specs=[
            pl.BlockSpec((B, F), lambda: (0, 0)),
            pl.BlockSpec((B, F), lambda: (0, 0)),
            pl.BlockSpec((B, 1), lambda: (0, 0)),
            pl.BlockSpec(memory_space=pl.ANY),
        ],
        out_specs=pl.BlockSpec(memory_space=pl.ANY),
        out_shape=jax.ShapeDtypeStruct((F, B, C), input_features.dtype),
        scratch_shapes=(
            [pltpu.VMEM((F, B), jnp.float32)]
            + [pltpu.VMEM((FBLK, B, C), jnp.float32) for _ in range(10)]
            + [pltpu.SemaphoreType.DMA for _ in range(10)]
        ),
    )(scores, mask_i32, aug_u, xt)
    return jnp.transpose(out_t, (1, 0, 2))
